# Initial kernel scaffold; baseline (speedup 1.0000x reference)
import jax
import jax.numpy as jnp
from jax.experimental import pallas as pl


def kernel(x, edge_index, edge_weight, W0, b0, gamma0, beta0, W1, b1, gamma1, beta1, W2, b2):
    raise NotImplementedError("write your pallas kernel here")


# speedup vs baseline: 5.6352x; 5.6352x over previous
import functools

import jax
import jax.numpy as jnp
from jax import lax
from jax.experimental import pallas as pl
from jax.experimental.pallas import tpu as pltpu
from jax.experimental.pallas import tpu_sc as plsc

N_NODES = 10000
D = 128
N_EDGES = 320000

NW = 32
EPW = N_EDGES // NW
CHUNK = 80
NCHUNK = EPW // CHUNK
SB = 25
NSB = NCHUNK // SB
GROUPS = CHUNK // 16
NPAD = 10240
STRIPE = NPAD // 16
WB = CHUNK
LANES = 16
SEGS = D // LANES


def _sc_agg_body(h_hbm, src_hbm, dst_hbm, ew_hbm, out_hbm,
                 src_v, dst_v, ew_v, rows_v, acc_sh, gsem, ssem):
    c = lax.axis_index("c")
    s = lax.axis_index("s")
    wid = c * 16 + s

    def _zrow(e, _):
        for j in range(SEGS):
            rows_v[e, pl.ds(j * LANES, LANES)] = jnp.zeros((LANES,), jnp.float32)
        return 0
    lax.fori_loop(0, WB, _zrow, 0)
    base = s * STRIPE
    for k in range(STRIPE // WB):
        pltpu.sync_copy(rows_v, acc_sh.at[pl.ds(base + k * WB, WB)])
    plsc.subcore_barrier()

    def _sblock(sb, _):
        pltpu.sync_copy(src_hbm.at[wid, sb], src_v)
        pltpu.sync_copy(dst_hbm.at[wid, sb], dst_v)
        pltpu.sync_copy(ew_hbm.at[wid, sb], ew_v)

        def _chunk(ci, _):
            pltpu.async_copy(h_hbm.at[src_v.at[ci]], rows_v, gsem).wait()

            def _scale(g, _):
                vw = ew_v[ci, pl.ds(g * LANES, LANES)]
                e0 = g * LANES
                for l in range(LANES):
                    w = vw[l]
                    for j in range(SEGS):
                        sl = pl.ds(j * LANES, LANES)
                        rows_v[e0 + l, sl] = rows_v[e0 + l, sl] * w
                return 0
            lax.fori_loop(0, GROUPS, _scale, 0)

            pltpu.async_copy(rows_v, acc_sh.at[dst_v.at[ci]], ssem,
                             add=True).wait()
            return 0
        lax.fori_loop(0, SB, _chunk, 0)
        return 0
    lax.fori_loop(0, NSB, _sblock, 0)
    plsc.subcore_barrier()

    for k in range(STRIPE // WB):
        r0 = base + k * WB
        pltpu.sync_copy(acc_sh.at[pl.ds(r0, WB)], rows_v)
        pltpu.sync_copy(rows_v, out_hbm.at[c, pl.ds(r0, WB)])


@functools.cache
def _get_sc_agg():
  return pl.kernel(
    _sc_agg_body,
    out_type=jax.ShapeDtypeStruct((2, NPAD, D), jnp.float32),
    mesh=plsc.VectorSubcoreMesh(core_axis_name="c", subcore_axis_name="s"),
    scratch_types=[
        pltpu.VMEM((SB, CHUNK), jnp.int32),
        pltpu.VMEM((SB, CHUNK), jnp.int32),
        pltpu.VMEM((SB, CHUNK), jnp.float32),
        pltpu.VMEM((CHUNK, D), jnp.float32),
        pltpu.VMEM_SHARED((NPAD, D), jnp.float32),
        pltpu.SemaphoreType.DMA,
        pltpu.SemaphoreType.DMA,
    ],
  )


_BLK = 1000


def _mm_body(x_ref, w_ref, o_ref):
    o_ref[...] = jnp.dot(x_ref[...], w_ref[...],
                         preferred_element_type=jnp.float32)


@jax.jit
def _tc_matmul(x, w):
    return pl.pallas_call(
        _mm_body,
        grid=(N_NODES // _BLK,),
        in_specs=[
            pl.BlockSpec((_BLK, D), lambda i: (i, 0)),
            pl.BlockSpec((D, D), lambda i: (0, 0)),
        ],
        out_specs=pl.BlockSpec((_BLK, D), lambda i: (i, 0)),
        out_shape=jax.ShapeDtypeStruct((N_NODES, D), jnp.float32),
    )(x, w)


def _comb_mm_body(p_ref, c_ref, w_ref, a_ref, h_ref):
    a = jnp.maximum(p_ref[0] + p_ref[1] + c_ref[...], 0.0)
    a_ref[...] = a
    h_ref[...] = jnp.dot(a, w_ref[...], preferred_element_type=jnp.float32)


@jax.jit
def _tc_combine_mm(p, cvec, w):
    return pl.pallas_call(
        _comb_mm_body,
        grid=(N_NODES // _BLK,),
        in_specs=[
            pl.BlockSpec((2, _BLK, D), lambda i: (0, i, 0)),
            pl.BlockSpec((1, D), lambda i: (0, 0)),
            pl.BlockSpec((D, D), lambda i: (0, 0)),
        ],
        out_specs=[
            pl.BlockSpec((_BLK, D), lambda i: (i, 0)),
            pl.BlockSpec((_BLK, D), lambda i: (i, 0)),
        ],
        out_shape=[
            jax.ShapeDtypeStruct((N_NODES, D), jnp.float32),
            jax.ShapeDtypeStruct((N_NODES, D), jnp.float32),
        ],
    )(p, cvec, w)


def _comb_body(p_ref, c_ref, o_ref):
    o_ref[...] = p_ref[0] + p_ref[1] + c_ref[...]


@jax.jit
def _tc_combine(p, cvec):
    return pl.pallas_call(
        _comb_body,
        grid=(N_NODES // _BLK,),
        in_specs=[
            pl.BlockSpec((2, _BLK, D), lambda i: (0, i, 0)),
            pl.BlockSpec((1, D), lambda i: (0, 0)),
        ],
        out_specs=pl.BlockSpec((_BLK, D), lambda i: (i, 0)),
        out_shape=jax.ShapeDtypeStruct((N_NODES, D), jnp.float32),
    )(p, cvec)


def kernel(x, edge_index, edge_weight, W0, b0, gamma0, beta0,
           W1, b1, gamma1, beta1, W2, b2):
    eps = 1e-5
    s0 = gamma0 / jnp.sqrt(1.0 + eps)
    s1 = gamma1 / jnp.sqrt(1.0 + eps)
    W0s = W0 * s0[None, :]
    W1s = W1 * s1[None, :]
    c0 = (b0 * s0 + beta0).reshape(1, D)
    c1 = (b1 * s1 + beta1).reshape(1, D)
    c2 = b2.reshape(1, D)

    src = edge_index[0].astype(jnp.int32).reshape(NW, NSB, SB, CHUNK)
    dst = edge_index[1].astype(jnp.int32).reshape(NW, NSB, SB, CHUNK)
    ew = edge_weight.astype(jnp.float32).reshape(NW, NSB, SB, CHUNK)

    sc_agg = _get_sc_agg()
    h0 = _tc_matmul(x, W0s)
    p0 = sc_agg(h0, src, dst, ew)
    _, h1 = _tc_combine_mm(p0, c0, W1s)
    p1 = sc_agg(h1, src, dst, ew)
    a2, h2 = _tc_combine_mm(p1, c1, W2)
    p2 = sc_agg(h2, src, dst, ew)
    logit = _tc_combine(p2, c2)
    return (logit, a2)
